# baseline (speedup 1.0000x reference)
import functools

import jax
import jax.numpy as jnp
from jax import lax
from jax.experimental import pallas as pl
from jax.experimental.pallas import tpu as pltpu
from jax.experimental.pallas import tpu_sc as plsc

VOCAB = 100000
EMBED = 128
HIDDEN = 256
NCLASS = 20
B = 1024
L = 200

NC = 2
NS = 16
NW = NC * NS
ROWS_PER_W = B * L // NW
CHUNK = 128
NCHUNK = ROWS_PER_W // CHUNK
NBUF = 2
FC_PAD = 128


def _gather_sc(emb_table, idx3d):
    mesh = plsc.VectorSubcoreMesh(core_axis_name="c", subcore_axis_name="s")

    @functools.partial(
        pl.kernel,
        mesh=mesh,
        out_type=jax.ShapeDtypeStruct((L * B, EMBED), jnp.float32),
        scratch_types=[
            pltpu.VMEM((NCHUNK, CHUNK), jnp.int32),
            pltpu.VMEM((NBUF, CHUNK, EMBED), jnp.float32),
            pltpu.SemaphoreType.DMA((NBUF,)),
        ],
    )
    def gather_kernel(table_hbm, idx_hbm, out_hbm, idx_v, rows_v, sems):
        wid = lax.axis_index("s") * NC + lax.axis_index("c")
        base = wid * ROWS_PER_W
        pltpu.sync_copy(idx_hbm.at[wid], idx_v)

        for b in range(NBUF):
            pltpu.async_copy(
                table_hbm.at[idx_v.at[b]], rows_v.at[b], sems.at[b])

        @pl.loop(0, NCHUNK, step=NBUF)
        def _chunks(j):
            for b in range(NBUF):
                pltpu.make_async_copy(
                    table_hbm.at[idx_v.at[j + b]], rows_v.at[b],
                    sems.at[b]).wait()
                pltpu.sync_copy(
                    rows_v.at[b],
                    out_hbm.at[pl.ds(base + (j + b) * CHUNK, CHUNK)])

                @pl.when(j + b + NBUF < NCHUNK)
                def _():
                    pltpu.async_copy(
                        table_hbm.at[idx_v.at[j + b + NBUF]], rows_v.at[b],
                        sems.at[b])

    return gather_kernel(emb_table, idx3d)


RB = 256
NRB = B // RB
TS = 20
NG = L // TS


def _rnn_body(xe_ref, wih_ref, whh_ref, b_ref, fcw_ref, fcb_ref,
              out_ref, h_scr):
    g = pl.program_id(0)

    @pl.when(g == 0)
    def _():
        h_scr[...] = jnp.zeros_like(h_scr)

    for s in range(TS):
        for i in range(NRB):
            rows = pl.ds(i * RB, RB)
            acc = jnp.dot(xe_ref[s, rows, :].astype(jnp.bfloat16),
                          wih_ref[...], preferred_element_type=jnp.float32)
            acc = acc + jnp.dot(h_scr[rows, :], whh_ref[...],
                                preferred_element_type=jnp.float32)
            h_scr[rows, :] = jnp.tanh(acc + b_ref[...]).astype(jnp.bfloat16)

    @pl.when(g == NG - 1)
    def _():
        out_ref[...] = (
            jnp.dot(h_scr[...], fcw_ref[...],
                    preferred_element_type=jnp.float32)
            + fcb_ref[...])


def _rnn_tc(xe, wih_t, whh_t, bias, fcw_t, fcb_p):
    return pl.pallas_call(
        _rnn_body,
        grid=(NG,),
        in_specs=[
            pl.BlockSpec((TS, B, EMBED), lambda t: (t, 0, 0)),
            pl.BlockSpec((EMBED, HIDDEN), lambda t: (0, 0)),
            pl.BlockSpec((HIDDEN, HIDDEN), lambda t: (0, 0)),
            pl.BlockSpec((1, HIDDEN), lambda t: (0, 0)),
            pl.BlockSpec((HIDDEN, FC_PAD), lambda t: (0, 0)),
            pl.BlockSpec((1, FC_PAD), lambda t: (0, 0)),
        ],
        out_specs=pl.BlockSpec((B, FC_PAD), lambda t: (0, 0)),
        out_shape=jax.ShapeDtypeStruct((B, FC_PAD), jnp.float32),
        scratch_shapes=[pltpu.VMEM((B, HIDDEN), jnp.bfloat16)],
    )(xe, wih_t, whh_t, bias, fcw_t, fcb_p)


def kernel(x, emb_table, W_ih, W_hh, b_ih, b_hh, fc_W, fc_b):
    idx3d = x.T.reshape(NW, NCHUNK, CHUNK)
    xe = _gather_sc(emb_table, idx3d).reshape(L, B, EMBED)

    wih_t = W_ih.T.astype(jnp.bfloat16)
    whh_t = W_hh.T.astype(jnp.bfloat16)
    bias = (b_ih + b_hh).reshape(1, HIDDEN)
    fcw_t = (jnp.zeros((HIDDEN, FC_PAD), jnp.float32)
             .at[:, :NCLASS].set(fc_W.T).astype(jnp.bfloat16))
    fcb_p = jnp.zeros((1, FC_PAD), jnp.float32).at[0, :NCLASS].set(fc_b)

    logits = _rnn_tc(xe, wih_t, whh_t, bias, fcw_t, fcb_p)
    return logits[:, :NCLASS]


# speedup vs baseline: 1.0586x; 1.0586x over previous
import functools

import jax
import jax.numpy as jnp
from jax import lax
from jax.experimental import pallas as pl
from jax.experimental.pallas import tpu as pltpu
from jax.experimental.pallas import tpu_sc as plsc

VOCAB = 100000
EMBED = 128
HIDDEN = 256
NCLASS = 20
B = 1024
L = 200

SEG = 5
LSEG = L // SEG

NC = 2
NS = 16
NW = NC * NS
ROWS_PER_W = LSEG * B // NW
CHUNK = 128
NCHUNK = ROWS_PER_W // CHUNK
NBUF = 2
FC_PAD = 128

RB = 256
NRB = B // RB
TS = 8
NG = LSEG // TS


def _gather_sc(emb_table, idx3d):
    mesh = plsc.VectorSubcoreMesh(core_axis_name="c", subcore_axis_name="s")

    @functools.partial(
        pl.kernel,
        mesh=mesh,
        out_type=jax.ShapeDtypeStruct((LSEG * B, EMBED), jnp.float32),
        scratch_types=[
            pltpu.VMEM((NCHUNK, CHUNK), jnp.int32),
            pltpu.VMEM((NBUF, CHUNK, EMBED), jnp.float32),
            pltpu.SemaphoreType.DMA((NBUF,)),
        ],
    )
    def gather_kernel(table_hbm, idx_hbm, out_hbm, idx_v, rows_v, sems):
        wid = lax.axis_index("s") * NC + lax.axis_index("c")
        base = wid * ROWS_PER_W
        pltpu.sync_copy(idx_hbm.at[wid], idx_v)

        for b in range(NBUF):
            pltpu.async_copy(
                table_hbm.at[idx_v.at[b]], rows_v.at[b], sems.at[b])

        @pl.loop(0, NCHUNK, step=NBUF)
        def _chunks(j):
            for b in range(NBUF):
                pltpu.make_async_copy(
                    table_hbm.at[idx_v.at[j + b]], rows_v.at[b],
                    sems.at[b]).wait()
                pltpu.sync_copy(
                    rows_v.at[b],
                    out_hbm.at[pl.ds(base + (j + b) * CHUNK, CHUNK)])

                @pl.when(j + b + NBUF < NCHUNK)
                def _():
                    pltpu.async_copy(
                        table_hbm.at[idx_v.at[j + b + NBUF]], rows_v.at[b],
                        sems.at[b])

    return gather_kernel(emb_table, idx3d)


def _rnn_body(xe_ref, hin_ref, wih_ref, whh_ref, b_ref, fcw_ref, fcb_ref,
              hout_ref, out_ref, h_scr):
    g = pl.program_id(0)

    @pl.when(g == 0)
    def _():
        h_scr[...] = hin_ref[...]

    for s in range(TS):
        for i in range(NRB):
            rows = pl.ds(i * RB, RB)
            acc = jnp.dot(xe_ref[s, rows, :].astype(jnp.bfloat16),
                          wih_ref[...], preferred_element_type=jnp.float32)
            acc = acc + jnp.dot(h_scr[rows, :], whh_ref[...],
                                preferred_element_type=jnp.float32)
            h_scr[rows, :] = jnp.tanh(acc + b_ref[...]).astype(jnp.bfloat16)

    @pl.when(g == NG - 1)
    def _():
        hout_ref[...] = h_scr[...]
        out_ref[...] = (
            jnp.dot(h_scr[...], fcw_ref[...],
                    preferred_element_type=jnp.float32)
            + fcb_ref[...])


def _rnn_seg(xe, h_in, wih_t, whh_t, bias, fcw_t, fcb_p):
    return pl.pallas_call(
        _rnn_body,
        grid=(NG,),
        in_specs=[
            pl.BlockSpec((TS, B, EMBED), lambda t: (t, 0, 0)),
            pl.BlockSpec((B, HIDDEN), lambda t: (0, 0)),
            pl.BlockSpec((EMBED, HIDDEN), lambda t: (0, 0)),
            pl.BlockSpec((HIDDEN, HIDDEN), lambda t: (0, 0)),
            pl.BlockSpec((1, HIDDEN), lambda t: (0, 0)),
            pl.BlockSpec((HIDDEN, FC_PAD), lambda t: (0, 0)),
            pl.BlockSpec((1, FC_PAD), lambda t: (0, 0)),
        ],
        out_specs=[
            pl.BlockSpec((B, HIDDEN), lambda t: (0, 0)),
            pl.BlockSpec((B, FC_PAD), lambda t: (0, 0)),
        ],
        out_shape=[
            jax.ShapeDtypeStruct((B, HIDDEN), jnp.bfloat16),
            jax.ShapeDtypeStruct((B, FC_PAD), jnp.float32),
        ],
        scratch_shapes=[pltpu.VMEM((B, HIDDEN), jnp.bfloat16)],
    )(xe, h_in, wih_t, whh_t, bias, fcw_t, fcb_p)


def kernel(x, emb_table, W_ih, W_hh, b_ih, b_hh, fc_W, fc_b):
    idx4d = x.T.reshape(SEG, NW, NCHUNK, CHUNK)

    wih_t = W_ih.T.astype(jnp.bfloat16)
    whh_t = W_hh.T.astype(jnp.bfloat16)
    bias = (b_ih + b_hh).reshape(1, HIDDEN)
    fcw_t = (jnp.zeros((HIDDEN, FC_PAD), jnp.float32)
             .at[:, :NCLASS].set(fc_W.T).astype(jnp.bfloat16))
    fcb_p = jnp.zeros((1, FC_PAD), jnp.float32).at[0, :NCLASS].set(fc_b)

    xes = [_gather_sc(emb_table, idx4d[s]) for s in range(SEG)]

    h = jnp.zeros((B, HIDDEN), jnp.bfloat16)
    logits = None
    for s in range(SEG):
        h, logits = _rnn_seg(xes[s].reshape(LSEG, B, EMBED), h,
                             wih_t, whh_t, bias, fcw_t, fcb_p)
    return logits[:, :NCLASS]


# speedup vs baseline: 1.0838x; 1.0237x over previous
import functools

import jax
import jax.numpy as jnp
from jax import lax
from jax.experimental import pallas as pl
from jax.experimental.pallas import tpu as pltpu
from jax.experimental.pallas import tpu_sc as plsc

VOCAB = 100000
EMBED = 128
HIDDEN = 256
NCLASS = 20
B = 1024
L = 200

SEG = 5
LSEG = L // SEG

NC = 2
NS = 16
NW = NC * NS
ROWS_PER_W = LSEG * B // NW
CHUNK = 128
NCHUNK = ROWS_PER_W // CHUNK
NBUF = 5
FC_PAD = 128

RB = 256
NRB = B // RB
TS = 8
NG = LSEG // TS


def _gather_sc(emb_table, idx3d):
    mesh = plsc.VectorSubcoreMesh(core_axis_name="c", subcore_axis_name="s")

    @functools.partial(
        pl.kernel,
        mesh=mesh,
        out_type=jax.ShapeDtypeStruct((LSEG * B, EMBED), jnp.float32),
        scratch_types=[
            pltpu.VMEM((NCHUNK, CHUNK), jnp.int32),
            pltpu.VMEM((NBUF, CHUNK, EMBED), jnp.float32),
            pltpu.SemaphoreType.DMA((NBUF,)),
            pltpu.SemaphoreType.DMA((NBUF,)),
        ],
    )
    def gather_kernel(table_hbm, idx_hbm, out_hbm, idx_v, rows_v, gsems,
                      wsems):
        wid = lax.axis_index("s") * NC + lax.axis_index("c")
        base = wid * ROWS_PER_W
        pltpu.sync_copy(idx_hbm.at[wid], idx_v)

        for b in range(NBUF):
            pltpu.async_copy(
                table_hbm.at[idx_v.at[b]], rows_v.at[b], gsems.at[b])

        @pl.loop(0, NCHUNK, step=NBUF)
        def _chunks(j):
            for b in range(NBUF):
                pltpu.make_async_copy(
                    table_hbm.at[idx_v.at[j + b]], rows_v.at[b],
                    gsems.at[b]).wait()
                pltpu.async_copy(
                    rows_v.at[b],
                    out_hbm.at[pl.ds(base + (j + b) * CHUNK, CHUNK)],
                    wsems.at[b])

                @pl.when(j + b + NBUF < NCHUNK)
                def _():
                    pltpu.make_async_copy(
                        rows_v.at[b],
                        out_hbm.at[pl.ds(base + (j + b) * CHUNK, CHUNK)],
                        wsems.at[b]).wait()
                    pltpu.async_copy(
                        table_hbm.at[idx_v.at[j + b + NBUF]], rows_v.at[b],
                        gsems.at[b])

        for b in range(NBUF):
            k = NCHUNK - NBUF + b
            pltpu.make_async_copy(
                rows_v.at[b],
                out_hbm.at[pl.ds(base + k * CHUNK, CHUNK)],
                wsems.at[b]).wait()

    return gather_kernel(emb_table, idx3d)


def _rnn_body(xe_ref, hin_ref, wih_ref, whh_ref, b_ref, fcw_ref, fcb_ref,
              hout_ref, out_ref, h_scr):
    g = pl.program_id(0)

    @pl.when(g == 0)
    def _():
        h_scr[...] = hin_ref[...]

    for s in range(TS):
        for i in range(NRB):
            rows = pl.ds(i * RB, RB)
            acc = jnp.dot(xe_ref[s, rows, :].astype(jnp.bfloat16),
                          wih_ref[...], preferred_element_type=jnp.float32)
            acc = acc + jnp.dot(h_scr[rows, :], whh_ref[...],
                                preferred_element_type=jnp.float32)
            h_scr[rows, :] = jnp.tanh(acc + b_ref[...]).astype(jnp.bfloat16)

    @pl.when(g == NG - 1)
    def _():
        hout_ref[...] = h_scr[...]
        out_ref[...] = (
            jnp.dot(h_scr[...], fcw_ref[...],
                    preferred_element_type=jnp.float32)
            + fcb_ref[...])


def _rnn_seg(xe, h_in, wih_t, whh_t, bias, fcw_t, fcb_p):
    return pl.pallas_call(
        _rnn_body,
        grid=(NG,),
        in_specs=[
            pl.BlockSpec((TS, B, EMBED), lambda t: (t, 0, 0)),
            pl.BlockSpec((B, HIDDEN), lambda t: (0, 0)),
            pl.BlockSpec((EMBED, HIDDEN), lambda t: (0, 0)),
            pl.BlockSpec((HIDDEN, HIDDEN), lambda t: (0, 0)),
            pl.BlockSpec((1, HIDDEN), lambda t: (0, 0)),
            pl.BlockSpec((HIDDEN, FC_PAD), lambda t: (0, 0)),
            pl.BlockSpec((1, FC_PAD), lambda t: (0, 0)),
        ],
        out_specs=[
            pl.BlockSpec((B, HIDDEN), lambda t: (0, 0)),
            pl.BlockSpec((B, FC_PAD), lambda t: (0, 0)),
        ],
        out_shape=[
            jax.ShapeDtypeStruct((B, HIDDEN), jnp.bfloat16),
            jax.ShapeDtypeStruct((B, FC_PAD), jnp.float32),
        ],
        scratch_shapes=[pltpu.VMEM((B, HIDDEN), jnp.bfloat16)],
    )(xe, h_in, wih_t, whh_t, bias, fcw_t, fcb_p)


def kernel(x, emb_table, W_ih, W_hh, b_ih, b_hh, fc_W, fc_b):
    idx4d = x.T.reshape(SEG, NW, NCHUNK, CHUNK)

    wih_t = W_ih.T.astype(jnp.bfloat16)
    whh_t = W_hh.T.astype(jnp.bfloat16)
    bias = (b_ih + b_hh).reshape(1, HIDDEN)
    fcw_t = (jnp.zeros((HIDDEN, FC_PAD), jnp.float32)
             .at[:, :NCLASS].set(fc_W.T).astype(jnp.bfloat16))
    fcb_p = jnp.zeros((1, FC_PAD), jnp.float32).at[0, :NCLASS].set(fc_b)

    xes = [_gather_sc(emb_table, idx4d[s]) for s in range(SEG)]

    h = jnp.zeros((B, HIDDEN), jnp.bfloat16)
    logits = None
    for s in range(SEG):
        h, logits = _rnn_seg(xes[s].reshape(LSEG, B, EMBED), h,
                             wih_t, whh_t, bias, fcw_t, fcb_p)
    return logits[:, :NCLASS]


# speedup vs baseline: 1.1538x; 1.0646x over previous
import functools

import jax
import jax.numpy as jnp
from jax import lax
from jax.experimental import pallas as pl
from jax.experimental.pallas import tpu as pltpu
from jax.experimental.pallas import tpu_sc as plsc

VOCAB = 100000
EMBED = 128
HIDDEN = 256
NCLASS = 20
B = 1024
L = 200

SEG = 5
LSEG = L // SEG

NC = 2
NS = 16
NW = NC * NS
ROWS_PER_W = LSEG * B // NW
CHUNK = 128
NCHUNK = ROWS_PER_W // CHUNK
NBUF = 5
FC_PAD = 128

RB = 256
NRB = B // RB
TS = 20
NG = LSEG // TS


def _gather_sc(emb_table, idx3d):
    mesh = plsc.VectorSubcoreMesh(core_axis_name="c", subcore_axis_name="s")

    @functools.partial(
        pl.kernel,
        mesh=mesh,
        out_type=jax.ShapeDtypeStruct((LSEG * B, EMBED), jnp.float32),
        scratch_types=[
            pltpu.VMEM((NCHUNK, CHUNK), jnp.int32),
            pltpu.VMEM((NBUF, CHUNK, EMBED), jnp.float32),
            pltpu.SemaphoreType.DMA((NBUF,)),
            pltpu.SemaphoreType.DMA((NBUF,)),
        ],
    )
    def gather_kernel(table_hbm, idx_hbm, out_hbm, idx_v, rows_v, gsems,
                      wsems):
        wid = lax.axis_index("s") * NC + lax.axis_index("c")
        base = wid * ROWS_PER_W
        pltpu.sync_copy(idx_hbm.at[wid], idx_v)

        for b in range(NBUF):
            pltpu.async_copy(
                table_hbm.at[idx_v.at[b]], rows_v.at[b], gsems.at[b])

        @pl.loop(0, NCHUNK, step=NBUF)
        def _chunks(j):
            for b in range(NBUF):
                pltpu.make_async_copy(
                    table_hbm.at[idx_v.at[j + b]], rows_v.at[b],
                    gsems.at[b]).wait()
                pltpu.async_copy(
                    rows_v.at[b],
                    out_hbm.at[pl.ds(base + (j + b) * CHUNK, CHUNK)],
                    wsems.at[b])

                @pl.when(j + b + NBUF < NCHUNK)
                def _():
                    pltpu.make_async_copy(
                        rows_v.at[b],
                        out_hbm.at[pl.ds(base + (j + b) * CHUNK, CHUNK)],
                        wsems.at[b]).wait()
                    pltpu.async_copy(
                        table_hbm.at[idx_v.at[j + b + NBUF]], rows_v.at[b],
                        gsems.at[b])

        for b in range(NBUF):
            k = NCHUNK - NBUF + b
            pltpu.make_async_copy(
                rows_v.at[b],
                out_hbm.at[pl.ds(base + k * CHUNK, CHUNK)],
                wsems.at[b]).wait()

    return gather_kernel(emb_table, idx3d)


def _rnn_body(xe_ref, hin_ref, wih_ref, whh_ref, b_ref, fcw_ref, fcb_ref,
              hout_ref, out_ref, h_scr):
    g = pl.program_id(0)

    @pl.when(g == 0)
    def _():
        h_scr[...] = hin_ref[...]

    for s in range(TS):
        for i in range(NRB):
            rows = pl.ds(i * RB, RB)
            acc = jnp.dot(xe_ref[s, rows, :].astype(jnp.bfloat16),
                          wih_ref[...], preferred_element_type=jnp.float32)
            acc = acc + jnp.dot(h_scr[rows, :], whh_ref[...],
                                preferred_element_type=jnp.float32)
            h_scr[rows, :] = jnp.tanh(acc + b_ref[...]).astype(jnp.bfloat16)

    @pl.when(g == NG - 1)
    def _():
        hout_ref[...] = h_scr[...]
        out_ref[...] = (
            jnp.dot(h_scr[...], fcw_ref[...],
                    preferred_element_type=jnp.float32)
            + fcb_ref[...])


def _rnn_seg(xe, h_in, wih_t, whh_t, bias, fcw_t, fcb_p):
    return pl.pallas_call(
        _rnn_body,
        grid=(NG,),
        in_specs=[
            pl.BlockSpec((TS, B, EMBED), lambda t: (t, 0, 0)),
            pl.BlockSpec((B, HIDDEN), lambda t: (0, 0)),
            pl.BlockSpec((EMBED, HIDDEN), lambda t: (0, 0)),
            pl.BlockSpec((HIDDEN, HIDDEN), lambda t: (0, 0)),
            pl.BlockSpec((1, HIDDEN), lambda t: (0, 0)),
            pl.BlockSpec((HIDDEN, FC_PAD), lambda t: (0, 0)),
            pl.BlockSpec((1, FC_PAD), lambda t: (0, 0)),
        ],
        out_specs=[
            pl.BlockSpec((B, HIDDEN), lambda t: (0, 0)),
            pl.BlockSpec((B, FC_PAD), lambda t: (0, 0)),
        ],
        out_shape=[
            jax.ShapeDtypeStruct((B, HIDDEN), jnp.bfloat16),
            jax.ShapeDtypeStruct((B, FC_PAD), jnp.float32),
        ],
        scratch_shapes=[pltpu.VMEM((B, HIDDEN), jnp.bfloat16)],
    )(xe, h_in, wih_t, whh_t, bias, fcw_t, fcb_p)


def kernel(x, emb_table, W_ih, W_hh, b_ih, b_hh, fc_W, fc_b):
    idx4d = x.T.reshape(SEG, NW, NCHUNK, CHUNK)

    wih_t = W_ih.T.astype(jnp.bfloat16)
    whh_t = W_hh.T.astype(jnp.bfloat16)
    bias = (b_ih + b_hh).reshape(1, HIDDEN)
    fcw_t = (jnp.zeros((HIDDEN, FC_PAD), jnp.float32)
             .at[:, :NCLASS].set(fc_W.T).astype(jnp.bfloat16))
    fcb_p = jnp.zeros((1, FC_PAD), jnp.float32).at[0, :NCLASS].set(fc_b)

    xes = [_gather_sc(emb_table, idx4d[s]) for s in range(SEG)]

    h = jnp.zeros((B, HIDDEN), jnp.bfloat16)
    logits = None
    for s in range(SEG):
        h, logits = _rnn_seg(xes[s].reshape(LSEG, B, EMBED), h,
                             wih_t, whh_t, bias, fcw_t, fcb_p)
    return logits[:, :NCLASS]
